# baseline (speedup 1.0000x reference)
import jax
import jax.numpy as jnp
import numpy as np
from jax import lax
from jax.experimental import pallas as pl
from jax.experimental.pallas import tpu as pltpu
from jax.experimental.pallas import tpu_sc as plsc

EMBED = 128
HIDDEN = 1024
BATCH = 16384
NC = 2
NS = 16
NW = NC * NS
B_PER_W = BATCH // NW
UNIT = 128
UNITS_PER_TAB = B_PER_W // UNIT
NUNITS = 3 * UNITS_PER_TAB
NBUF = 4

_PERM = np.empty(3 * EMBED, dtype=np.int32)
for _g in range(3 * EMBED // 32):
    _base = 32 * _g
    for _i in range(16):
        _PERM[_base + 2 * _i] = _base + _i
        _PERM[_base + 2 * _i + 1] = _base + 16 + _i


def _gather_body(xp, xo, xs, wp, wo, ws, op, oo, osub,
                 i0, i1, i2, f0, f1, f2, f3, c0, c1, c2, c3, gsem, wsem):
    wid = lax.axis_index("s") * NC + lax.axis_index("c")
    base = wid * B_PER_W
    idxs = (i0, i1, i2)
    tabs = (wp, wo, ws)
    outs = (op, oo, osub)
    fbufs = (f0, f1, f2, f3)
    cbufs = (c0, c1, c2, c3)
    for x_hbm, iv in zip((xp, xo, xs), idxs):
        pltpu.sync_copy(x_hbm.at[pl.ds(base, B_PER_W)], iv)

    def g(u):
        t, j = divmod(u, UNITS_PER_TAB)
        return (tabs[t].at[idxs[t].at[pl.ds(j * UNIT, UNIT)]],
                fbufs[u % NBUF])

    def w(u):
        t, j = divmod(u, UNITS_PER_TAB)
        return (cbufs[u % NBUF], outs[t].at[pl.ds(base + j * UNIT, UNIT)])

    def pack_unit(fb, cb):
        def row(r, carry):
            for c in (0, 32, 64, 96):
                a = jnp.maximum(fb[r, pl.ds(c, 16)], 0.0)
                b = jnp.maximum(fb[r, pl.ds(c + 16, 16)], 0.0)
                cb[r, pl.ds(c, 32)] = plsc.pack(
                    a, b, format=plsc.PackFormat.INTERLEAVED)
            return carry
        lax.fori_loop(0, UNIT, row, 0)

    for u in range(NBUF):
        pltpu.async_copy(*g(u), gsem)
    for u in range(NUNITS):
        pltpu.make_async_copy(*g(u), gsem).wait()
        pack_unit(fbufs[u % NBUF], cbufs[u % NBUF])
        pltpu.async_copy(*w(u), wsem)
        if u + NBUF < NUNITS:
            pltpu.make_async_copy(*w(u), wsem).wait()
            pltpu.async_copy(*g(u + NBUF), gsem)
    for u in range(NUNITS - NBUF, NUNITS):
        pltpu.make_async_copy(*w(u), wsem).wait()


_h_type = jax.ShapeDtypeStruct((BATCH, EMBED), jnp.bfloat16)

_gather = pl.kernel(
    _gather_body,
    mesh=plsc.VectorSubcoreMesh(core_axis_name="c", subcore_axis_name="s"),
    out_type=(_h_type, _h_type, _h_type),
    scratch_types=(
        [pltpu.VMEM((B_PER_W,), jnp.int32)] * 3
        + [pltpu.VMEM((UNIT, EMBED), jnp.float32)] * NBUF
        + [pltpu.VMEM((UNIT, EMBED), jnp.bfloat16)] * NBUF
        + [pltpu.SemaphoreType.DMA, pltpu.SemaphoreType.DMA]
    ),
    compiler_params=pltpu.CompilerParams(needs_layout_passes=False),
)


BM = 4096


def _mlp_body(hp, ho, hs, w, b, o):
    h = jnp.concatenate((hp[...], ho[...], hs[...]), axis=1)
    acc = jnp.dot(h, w[...], preferred_element_type=jnp.float32)
    o[...] = jnp.maximum(acc + b[...], 0.0)


def _mlp(hp, ho, hs, w, b):
    return pl.pallas_call(
        _mlp_body,
        grid=(BATCH // BM,),
        in_specs=[
            pl.BlockSpec((BM, EMBED), lambda i: (i, 0)),
            pl.BlockSpec((BM, EMBED), lambda i: (i, 0)),
            pl.BlockSpec((BM, EMBED), lambda i: (i, 0)),
            pl.BlockSpec((3 * EMBED, HIDDEN), lambda i: (0, 0)),
            pl.BlockSpec((1, HIDDEN), lambda i: (0, 0)),
        ],
        out_specs=pl.BlockSpec((BM, HIDDEN), lambda i: (i, 0)),
        out_shape=jax.ShapeDtypeStruct((BATCH, HIDDEN), jnp.float32),
    )(hp, ho, hs, w, b)


def kernel(X_phase, X_occurrence, X_subject, X_lengths,
           W_phase, W_occurrence, W_subject, W_out, b_out):
    del X_lengths
    hp, ho, hs = _gather(
        X_phase.astype(jnp.int32),
        X_occurrence.astype(jnp.int32),
        X_subject.astype(jnp.int32),
        W_phase, W_occurrence, W_subject,
    )
    w_perm = W_out[jnp.asarray(_PERM)].astype(jnp.bfloat16)
    return _mlp(hp, ho, hs, w_perm, b_out.reshape(1, HIDDEN))


# speedup vs baseline: 1.0766x; 1.0766x over previous
import jax
import jax.numpy as jnp
import numpy as np
from jax import lax
from jax.experimental import pallas as pl
from jax.experimental.pallas import tpu as pltpu
from jax.experimental.pallas import tpu_sc as plsc

EMBED = 128
HIDDEN = 1024
BATCH = 16384
NC = 2
NS = 16
NW = NC * NS
B_PER_W = BATCH // NW
UNIT = 128
UNITS_PER_TAB = B_PER_W // UNIT
NUNITS = 3 * UNITS_PER_TAB
NBUF = 4

_PERM = np.empty(3 * EMBED, dtype=np.int32)
for _g in range(3 * EMBED // 32):
    _base = 32 * _g
    for _i in range(16):
        _PERM[_base + 2 * _i] = _base + _i
        _PERM[_base + 2 * _i + 1] = _base + 16 + _i


def _gather_body(xp, xo, xs, wp, wo, ws, op, oo, osub,
                 i0, i1, i2, f0, f1, f2, f3, c0, c1, c2, c3, gsem, wsem):
    wid = lax.axis_index("s") * NC + lax.axis_index("c")
    base = wid * B_PER_W
    idxs = (i0, i1, i2)
    tabs = (wp, wo, ws)
    outs = (op, oo, osub)
    fbufs = (f0, f1, f2, f3)
    cbufs = (c0, c1, c2, c3)
    for x_hbm, iv in zip((xp, xo, xs), idxs):
        pltpu.sync_copy(x_hbm.at[pl.ds(base, B_PER_W)], iv)

    def g(u):
        t, j = divmod(u, UNITS_PER_TAB)
        return (tabs[t].at[idxs[t].at[pl.ds(j * UNIT, UNIT)]],
                fbufs[u % NBUF])

    def w(u):
        t, j = divmod(u, UNITS_PER_TAB)
        return (cbufs[u % NBUF], outs[t].at[pl.ds(base + j * UNIT, UNIT)])

    def pack_unit(fb, cb):
        def rows(r, carry):
            for c in (0, 32, 64, 96):
                cb[r, pl.ds(c, 32)] = plsc.pack(
                    fb[r, pl.ds(c, 16)],
                    fb[r, pl.ds(c + 16, 16)],
                    format=plsc.PackFormat.INTERLEAVED)
            return carry
        lax.fori_loop(0, UNIT, rows, 0)

    for u in range(NBUF):
        pltpu.async_copy(*g(u), gsem)
    for u in range(NUNITS):
        pltpu.make_async_copy(*g(u), gsem).wait()
        if u >= NBUF:
            pltpu.make_async_copy(*w(u - NBUF), wsem).wait()
        pack_unit(fbufs[u % NBUF], cbufs[u % NBUF])
        if u + NBUF < NUNITS:
            pltpu.async_copy(*g(u + NBUF), gsem)
        pltpu.async_copy(*w(u), wsem)
    for u in range(NUNITS - NBUF, NUNITS):
        pltpu.make_async_copy(*w(u), wsem).wait()


_h_type = jax.ShapeDtypeStruct((BATCH, EMBED), jnp.bfloat16)

_gather = pl.kernel(
    _gather_body,
    mesh=plsc.VectorSubcoreMesh(core_axis_name="c", subcore_axis_name="s"),
    out_type=(_h_type, _h_type, _h_type),
    scratch_types=(
        [pltpu.VMEM((B_PER_W,), jnp.int32)] * 3
        + [pltpu.VMEM((UNIT, EMBED), jnp.float32)] * NBUF
        + [pltpu.VMEM((UNIT, EMBED), jnp.bfloat16)] * NBUF
        + [pltpu.SemaphoreType.DMA, pltpu.SemaphoreType.DMA]
    ),
    compiler_params=pltpu.CompilerParams(needs_layout_passes=False),
)


BM = 4096


def _mlp_body(hp, ho, hs, w, b, o):
    h = jnp.concatenate((hp[...], ho[...], hs[...]), axis=1)
    h = jnp.maximum(h, jnp.bfloat16(0))
    acc = jnp.dot(h, w[...], preferred_element_type=jnp.float32)
    o[...] = jnp.maximum(acc + b[...], 0.0)


def _mlp(hp, ho, hs, w, b):
    return pl.pallas_call(
        _mlp_body,
        grid=(BATCH // BM,),
        in_specs=[
            pl.BlockSpec((BM, EMBED), lambda i: (i, 0)),
            pl.BlockSpec((BM, EMBED), lambda i: (i, 0)),
            pl.BlockSpec((BM, EMBED), lambda i: (i, 0)),
            pl.BlockSpec((3 * EMBED, HIDDEN), lambda i: (0, 0)),
            pl.BlockSpec((1, HIDDEN), lambda i: (0, 0)),
        ],
        out_specs=pl.BlockSpec((BM, HIDDEN), lambda i: (i, 0)),
        out_shape=jax.ShapeDtypeStruct((BATCH, HIDDEN), jnp.float32),
    )(hp, ho, hs, w, b)


def kernel(X_phase, X_occurrence, X_subject, X_lengths,
           W_phase, W_occurrence, W_subject, W_out, b_out):
    del X_lengths
    hp, ho, hs = _gather(
        X_phase.astype(jnp.int32),
        X_occurrence.astype(jnp.int32),
        X_subject.astype(jnp.int32),
        W_phase, W_occurrence, W_subject,
    )
    w_perm = W_out[jnp.asarray(_PERM)].astype(jnp.bfloat16)
    return _mlp(hp, ho, hs, w_perm, b_out.reshape(1, HIDDEN))
